# Initial kernel scaffold; baseline (speedup 1.0000x reference)
import jax
import jax.numpy as jnp
from jax.experimental import pallas as pl


def kernel(x, x_lens, y_padded, y_lens, W_enc, b_enc, embed, W_encp, b_encp, W_decp, b_decp, W_out, b_out):
    raise NotImplementedError("write your pallas kernel here")


# speedup vs baseline: 6.4077x; 6.4077x over previous
import jax
import jax.numpy as jnp
from jax.experimental import pallas as pl
from jax.experimental.pallas import tpu as pltpu

B, T, F = 4, 512, 80
ENC, DEC, JOIN, V = 512, 512, 512, 500
U = 32
UP1 = U + 1
BLANK = 0
TT = 128
NT = T // TT
D2 = T + U + 2
NEG = -1e30


def _stage1_kernel(x_ref, sos_ref, W_enc_ref, b_enc_ref, embed_ref,
                   W_encp_ref, b_encp_ref, W_decp_ref, b_decp_ref,
                   W_out_ref, b_out_ref, bl_ref, y_ref, dec_p_ref):
    tt = pl.program_id(1)

    @pl.when(tt == 0)
    def _init():
        bl_ref[...] = jnp.zeros_like(bl_ref)
        y_ref[...] = jnp.zeros_like(y_ref)
        sos = sos_ref[0, 0, :]
        oh = (jax.lax.broadcasted_iota(jnp.int32, (UP1, V), 1)
              == sos[:, None]).astype(jnp.float32)
        dec = jnp.tanh(jnp.dot(oh, embed_ref[...],
                               preferred_element_type=jnp.float32))
        dec_p_ref[...] = (jnp.dot(dec, W_decp_ref[...],
                                  preferred_element_type=jnp.float32)
                          + b_decp_ref[...])

    x = x_ref[0]
    enc = jnp.tanh(jnp.dot(x, W_enc_ref[...],
                           preferred_element_type=jnp.float32)
                   + b_enc_ref[...])
    enc_p = (jnp.dot(enc, W_encp_ref[...],
                     preferred_element_type=jnp.float32)
             + b_encp_ref[...])

    joint = jnp.tanh(enc_p[:, None, :] + dec_p_ref[...][None, :, :])
    logits = (jnp.dot(joint.reshape(TT * UP1, JOIN), W_out_ref[...],
                      preferred_element_type=jnp.float32)
              + b_out_ref[...]).reshape(TT, UP1, V)

    m = jnp.max(logits, axis=-1)
    s = jnp.sum(jnp.exp(logits - m[:, :, None]), axis=-1)
    lse = m + jnp.log(s)
    blank_lp = logits[:, :, BLANK] - lse

    sos = sos_ref[0, 0, :]
    ymask = (jax.lax.broadcasted_iota(jnp.int32, (UP1, V), 1)
             == sos[:, None]).astype(jnp.float32)
    y_logit = jnp.sum(logits[:, :U, :] * ymask[None, 1:, :], axis=-1)
    y_lp = y_logit - lse[:, :U]

    t0 = tt * TT
    for u in range(UP1):
        bl_ref[0, pl.ds(t0 + u, TT), u] = blank_lp[:, u]
    for j in range(U):
        y_ref[0, pl.ds(t0 + j + 1, TT), j + 1] = y_lp[:, j]


def _dp_kernel(bl_ref, y_ref, tl_ref, ul_ref, out_ref):
    u_iota = jax.lax.broadcasted_iota(jnp.int32, (B, UP1), 1)
    db = tl_ref[...] + ul_ref[...] - 1

    a0 = jnp.where(u_iota == 0, 0.0, NEG).astype(jnp.float32)
    cap_a = jnp.zeros((B, UP1), jnp.float32)
    cap_b = jnp.zeros((B, UP1), jnp.float32)

    def body(d, carry):
        a, ca, cb = carry
        blrow = bl_ref[:, pl.ds(d - 1, 1), :].reshape(B, UP1)
        yrow = y_ref[:, pl.ds(d, 1), :].reshape(B, UP1)
        p1 = a + blrow
        a_sh = jnp.concatenate(
            [jnp.full((B, 1), NEG, jnp.float32), a[:, :U]], axis=1)
        p2 = a_sh + yrow
        mx = jnp.maximum(p1, p2)
        mn = jnp.minimum(p1, p2)
        a_new = mx + jnp.log1p(jnp.exp(mn - mx))
        tval = d - u_iota
        valid = (tval >= 0) & (tval <= T - 1)
        a_new = jnp.where(valid, a_new, NEG)
        ca = jnp.where(db == d, a_new, ca)
        cb = jnp.where(db == d - 1, blrow, cb)
        return a_new, ca, cb

    _, cap_a, cap_b = jax.lax.fori_loop(1, T + U + 1, body,
                                        (a0, cap_a, cap_b))
    final = cap_a + cap_b
    sel = (u_iota == ul_ref[...]).astype(jnp.float32)
    out_ref[...] = (-jnp.sum(final * sel)).reshape(1, 1)


def kernel(x, x_lens, y_padded, y_lens, W_enc, b_enc, embed, W_encp, b_encp,
           W_decp, b_decp, W_out, b_out):
    sos = jnp.concatenate(
        [jnp.full((B, 1), BLANK, dtype=y_padded.dtype), y_padded],
        axis=1).astype(jnp.int32).reshape(B, 1, UP1)

    grid = (B, NT)
    bl_skew, y_skew = pl.pallas_call(
        _stage1_kernel,
        grid=grid,
        in_specs=[
            pl.BlockSpec((1, TT, F), lambda b, t: (b, t, 0)),
            pl.BlockSpec((1, 1, UP1), lambda b, t: (b, 0, 0)),
            pl.BlockSpec((F, ENC), lambda b, t: (0, 0)),
            pl.BlockSpec((1, ENC), lambda b, t: (0, 0)),
            pl.BlockSpec((V, DEC), lambda b, t: (0, 0)),
            pl.BlockSpec((ENC, JOIN), lambda b, t: (0, 0)),
            pl.BlockSpec((1, JOIN), lambda b, t: (0, 0)),
            pl.BlockSpec((DEC, JOIN), lambda b, t: (0, 0)),
            pl.BlockSpec((1, JOIN), lambda b, t: (0, 0)),
            pl.BlockSpec((JOIN, V), lambda b, t: (0, 0)),
            pl.BlockSpec((1, V), lambda b, t: (0, 0)),
        ],
        out_specs=[
            pl.BlockSpec((1, D2, UP1), lambda b, t: (b, 0, 0)),
            pl.BlockSpec((1, D2, UP1), lambda b, t: (b, 0, 0)),
        ],
        out_shape=[
            jax.ShapeDtypeStruct((B, D2, UP1), jnp.float32),
            jax.ShapeDtypeStruct((B, D2, UP1), jnp.float32),
        ],
        scratch_shapes=[pltpu.VMEM((UP1, JOIN), jnp.float32)],
        compiler_params=pltpu.CompilerParams(
            dimension_semantics=("arbitrary", "arbitrary")),
    )(x, sos, W_enc, b_enc.reshape(1, ENC), embed, W_encp,
      b_encp.reshape(1, JOIN), W_decp, b_decp.reshape(1, JOIN), W_out,
      b_out.reshape(1, V))

    nll = pl.pallas_call(
        _dp_kernel,
        out_shape=jax.ShapeDtypeStruct((1, 1), jnp.float32),
    )(bl_skew, y_skew,
      x_lens.astype(jnp.int32).reshape(B, 1),
      y_lens.astype(jnp.int32).reshape(B, 1))
    return nll.reshape(())


# speedup vs baseline: 10.8454x; 1.6925x over previous
import jax
import jax.numpy as jnp
from jax.experimental import pallas as pl
from jax.experimental.pallas import tpu as pltpu

B, T, F = 4, 512, 80
ENC, DEC, JOIN, V = 512, 512, 512, 500
U = 32
UP1 = U + 1
BLANK = 0
TT = 256
NT = T // TT
D2 = T + U + 2
NEG = -1e30

f32 = jnp.float32
bf16 = jnp.bfloat16


def _stage1_kernel(x_ref, sos_ref, W_enc_ref, b_enc_ref, embed_ref,
                   W_encp_ref, b_encp_ref, W_decp_ref, b_decp_ref,
                   W_out_ref, b_out_ref, bl_ref, y_ref,
                   dec_p_ref, Wy_ref, by_ref, expb_ref, S_ref, Bk_ref, Y_ref):
    tt = pl.program_id(1)

    @pl.when(tt == 0)
    def _init():
        bl_ref[...] = jnp.zeros_like(bl_ref)
        y_ref[...] = jnp.zeros_like(y_ref)
        expb_ref[...] = jnp.exp(b_out_ref[...]).astype(bf16)

        sos_row = sos_ref[0]
        sos_col = jnp.swapaxes(sos_row, 0, 1)
        oh = (jax.lax.broadcasted_iota(jnp.int32, (UP1, V), 1)
              == sos_col).astype(bf16)
        dec = jnp.tanh(jnp.dot(oh, embed_ref[...],
                               preferred_element_type=f32)).astype(bf16)
        dec_p = (jnp.dot(dec, W_decp_ref[...], preferred_element_type=f32)
                 + b_decp_ref[...])
        dec_p_ref[...] = dec_p.astype(bf16)

        labels = jnp.concatenate(
            [sos_row[:, 1:],
             jnp.full((1, 1), -1, jnp.int32),
             jnp.full((1, 1), BLANK, jnp.int32)], axis=1)
        oht = (jax.lax.broadcasted_iota(jnp.int32, (V, UP1 + 1), 0)
               == labels)
        Wy_ref[...] = jnp.dot(W_out_ref[...], oht.astype(bf16),
                              preferred_element_type=f32).astype(bf16)
        by_ref[...] = jnp.dot(b_out_ref[...], oht.astype(f32),
                              preferred_element_type=f32)

    xb = x_ref[0]
    h = jnp.dot(xb, W_enc_ref[...], preferred_element_type=f32) \
        + b_enc_ref[...]
    enc = jnp.tanh(h.astype(bf16))
    enc_p = (jnp.dot(enc, W_encp_ref[...], preferred_element_type=f32)
             + b_encp_ref[...])
    enc_pb = enc_p.astype(bf16)

    ones_col = jnp.ones((V, 1), bf16)
    for u in range(UP1):
        jt = jnp.tanh(enc_pb + dec_p_ref[u:u + 1, :])
        logits = jnp.dot(jt, W_out_ref[...],
                         preferred_element_type=f32).astype(bf16)
        el = jnp.exp(logits) * expb_ref[...]
        S_ref[:, u:u + 1] = jnp.dot(el, ones_col,
                                    preferred_element_type=f32)
        Bk_ref[:, u:u + 1] = jnp.dot(jt, Wy_ref[:, UP1:UP1 + 1],
                                     preferred_element_type=f32)
        Y_ref[:, u:u + 1] = jnp.dot(jt, Wy_ref[:, u:u + 1],
                                    preferred_element_type=f32)

    lse = jnp.log(S_ref[...])
    blank_lp = Bk_ref[...] + by_ref[0:1, UP1:UP1 + 1] - lse
    y_lp = Y_ref[...] + by_ref[0:1, :UP1] - lse

    t0 = tt * TT
    for u in range(UP1):
        bl_ref[0, pl.ds(t0 + u, TT), u] = blank_lp[:, u]
    for j in range(U):
        y_ref[0, pl.ds(t0 + j + 1, TT), j + 1] = y_lp[:, j]


def _dp_kernel(bl_ref, y_ref, tl_ref, ul_ref, out_ref):
    u_iota = jax.lax.broadcasted_iota(jnp.int32, (B, UP1), 1)
    db = tl_ref[...] + ul_ref[...] - 1

    a0 = jnp.where(u_iota == 0, 0.0, NEG).astype(f32)
    cap_a = jnp.zeros((B, UP1), f32)
    cap_b = jnp.zeros((B, UP1), f32)

    def body(d, carry):
        a, ca, cb = carry
        blrow = bl_ref[:, pl.ds(d - 1, 1), :].reshape(B, UP1)
        yrow = y_ref[:, pl.ds(d, 1), :].reshape(B, UP1)
        p1 = a + blrow
        a_sh = jnp.concatenate(
            [jnp.full((B, 1), NEG, f32), a[:, :U]], axis=1)
        p2 = a_sh + yrow
        mx = jnp.maximum(p1, p2)
        mn = jnp.minimum(p1, p2)
        a_new = mx + jnp.log1p(jnp.exp(mn - mx))
        tval = d - u_iota
        valid = (tval >= 0) & (tval <= T - 1)
        a_new = jnp.where(valid, a_new, NEG)
        ca = jnp.where(db == d, a_new, ca)
        cb = jnp.where(db == d - 1, blrow, cb)
        return a_new, ca, cb

    _, cap_a, cap_b = jax.lax.fori_loop(1, T + U + 1, body,
                                        (a0, cap_a, cap_b))
    final = cap_a + cap_b
    sel = (u_iota == ul_ref[...]).astype(f32)
    out_ref[...] = (-jnp.sum(final * sel)).reshape(1, 1)


def kernel(x, x_lens, y_padded, y_lens, W_enc, b_enc, embed, W_encp, b_encp,
           W_decp, b_decp, W_out, b_out):
    sos = jnp.concatenate(
        [jnp.full((B, 1), BLANK, dtype=y_padded.dtype), y_padded],
        axis=1).astype(jnp.int32).reshape(B, 1, UP1)

    grid = (B, NT)
    bl_skew, y_skew = pl.pallas_call(
        _stage1_kernel,
        grid=grid,
        in_specs=[
            pl.BlockSpec((1, TT, F), lambda b, t: (b, t, 0)),
            pl.BlockSpec((1, 1, UP1), lambda b, t: (b, 0, 0)),
            pl.BlockSpec((F, ENC), lambda b, t: (0, 0)),
            pl.BlockSpec((1, ENC), lambda b, t: (0, 0)),
            pl.BlockSpec((V, DEC), lambda b, t: (0, 0)),
            pl.BlockSpec((ENC, JOIN), lambda b, t: (0, 0)),
            pl.BlockSpec((1, JOIN), lambda b, t: (0, 0)),
            pl.BlockSpec((DEC, JOIN), lambda b, t: (0, 0)),
            pl.BlockSpec((1, JOIN), lambda b, t: (0, 0)),
            pl.BlockSpec((JOIN, V), lambda b, t: (0, 0)),
            pl.BlockSpec((1, V), lambda b, t: (0, 0)),
        ],
        out_specs=[
            pl.BlockSpec((1, D2, UP1), lambda b, t: (b, 0, 0)),
            pl.BlockSpec((1, D2, UP1), lambda b, t: (b, 0, 0)),
        ],
        out_shape=[
            jax.ShapeDtypeStruct((B, D2, UP1), f32),
            jax.ShapeDtypeStruct((B, D2, UP1), f32),
        ],
        scratch_shapes=[
            pltpu.VMEM((UP1, JOIN), bf16),
            pltpu.VMEM((JOIN, UP1 + 1), bf16),
            pltpu.VMEM((1, UP1 + 1), f32),
            pltpu.VMEM((1, V), bf16),
            pltpu.VMEM((TT, UP1), f32),
            pltpu.VMEM((TT, UP1), f32),
            pltpu.VMEM((TT, UP1), f32),
        ],
        compiler_params=pltpu.CompilerParams(
            dimension_semantics=("arbitrary", "arbitrary")),
    )(x.astype(bf16), sos, W_enc.astype(bf16), b_enc.reshape(1, ENC),
      embed.astype(bf16), W_encp.astype(bf16), b_encp.reshape(1, JOIN),
      W_decp.astype(bf16), b_decp.reshape(1, JOIN), W_out.astype(bf16),
      b_out.reshape(1, V))

    nll = pl.pallas_call(
        _dp_kernel,
        out_shape=jax.ShapeDtypeStruct((1, 1), f32),
    )(bl_skew, y_skew,
      x_lens.astype(jnp.int32).reshape(B, 1),
      y_lens.astype(jnp.int32).reshape(B, 1))
    return nll.reshape(())


# speedup vs baseline: 18.1374x; 1.6724x over previous
import jax
import jax.numpy as jnp
from jax.experimental import pallas as pl
from jax.experimental.pallas import tpu as pltpu

B, T, F = 4, 512, 80
ENC, DEC, JOIN, V = 512, 512, 512, 500
U = 32
UP1 = U + 1
BLANK = 0
TT = 256
NT = T // TT
D2 = T + U + 2
NEG = -1e30
UG = 11

f32 = jnp.float32
bf16 = jnp.bfloat16


def _stage1_kernel(x_ref, sos_ref, W_enc_ref, b_enc_ref, embed_ref,
                   W_encp_ref, b_encp_ref, W_decp_ref, b_decp_ref,
                   W_out_ref, b_out_ref, bl_ref, y_ref,
                   dec_p_ref, Wy_ref, by_ref, expb_ref, S_ref, Bk_ref, Y_ref,
                   Wext_ref):
    tt = pl.program_id(1)

    @pl.when(tt == 0)
    def _init():
        bl_ref[...] = jnp.full(bl_ref.shape, NEG, f32)
        y_ref[...] = jnp.full(y_ref.shape, NEG, f32)
        expb_ref[...] = jnp.exp(b_out_ref[...]).astype(bf16)

        sos_row = sos_ref[0]
        sos_col = jnp.swapaxes(sos_row, 0, 1)
        oh = (jax.lax.broadcasted_iota(jnp.int32, (UP1, V), 1)
              == sos_col).astype(bf16)
        dec = jnp.tanh(jnp.dot(oh, embed_ref[...],
                               preferred_element_type=f32)).astype(bf16)
        dec_p = (jnp.dot(dec, W_decp_ref[...], preferred_element_type=f32)
                 + b_decp_ref[...])
        dec_p_ref[...] = dec_p.astype(bf16)

        labels = jnp.concatenate(
            [sos_row[:, 1:],
             jnp.full((1, 4), -1, jnp.int32)], axis=1)
        oht = (jax.lax.broadcasted_iota(jnp.int32, (V, 36), 0)
               == labels)
        Wy_ref[...] = jnp.dot(W_out_ref[...], oht.astype(bf16),
                              preferred_element_type=f32).astype(bf16)
        by_ref[...] = jnp.dot(b_out_ref[...], oht.astype(f32),
                              preferred_element_type=f32)
        Wext_ref[:, :V] = W_out_ref[...]
        Wext_ref[:, V:] = jnp.zeros((JOIN, 12), bf16)

    xb = x_ref[0]
    h = jnp.dot(xb, W_enc_ref[...], preferred_element_type=f32) \
        + b_enc_ref[...]
    enc = jnp.tanh(h.astype(bf16))
    enc_p = (jnp.dot(enc, W_encp_ref[...], preferred_element_type=f32)
             + b_encp_ref[...])
    enc_pb = enc_p.astype(bf16)

    ones_col = jnp.ones((V, 1), bf16)
    for g in range(0, UP1, UG):
        Wext_ref[:, V:V + UG] = Wy_ref[:, g:g + UG]
        jts = [jnp.tanh(enc_pb + dec_p_ref[u:u + 1, :])
               for u in range(g, g + UG)]
        JT = jnp.concatenate(jts, axis=0)
        LO = jnp.dot(JT, Wext_ref[...],
                     preferred_element_type=f32)
        EL = jnp.exp(LO[:, :V].astype(bf16)) * expb_ref[...]
        SS = jnp.dot(EL, ones_col, preferred_element_type=f32)
        for i in range(UG):
            u = g + i
            S_ref[:, u:u + 1] = SS[i * TT:(i + 1) * TT]
            Bk_ref[:, u:u + 1] = LO[i * TT:(i + 1) * TT, BLANK:BLANK + 1]
            Y_ref[:, u:u + 1] = LO[i * TT:(i + 1) * TT, V + i:V + i + 1]

    lse = jnp.log(S_ref[...])
    blank_lp = Bk_ref[...] + b_out_ref[0:1, BLANK:BLANK + 1] - lse
    y_lp = Y_ref[...] + by_ref[0:1, :UP1] - lse

    t0 = tt * TT
    for u in range(UP1):
        bl_ref[0, pl.ds(t0 + u, TT), u] = blank_lp[:, u]
    for j in range(U):
        y_ref[0, pl.ds(t0 + j + 1, TT), j + 1] = y_lp[:, j]


def _dp_kernel(bl_ref, y_ref, tl_ref, ul_ref, out_ref, BLT_ref, YT_ref):
    BLT_ref[...] = jnp.stack([bl_ref[b] for b in range(B)], axis=-1)
    YT_ref[...] = jnp.stack([y_ref[b] for b in range(B)], axis=-1)

    u_iota = jax.lax.broadcasted_iota(jnp.int32, (UP1, B), 0)
    dbrow = tl_ref[...] + ul_ref[...] - 1

    a0 = jnp.where(u_iota == 0, 0.0, NEG).astype(f32)
    cap_a = jnp.zeros((UP1, B), f32)
    cap_b = jnp.zeros((UP1, B), f32)

    def body(d, carry):
        a, ca, cb = carry
        blrow = BLT_ref[pl.ds(d - 1, 1)].reshape(UP1, B)
        yrow = YT_ref[pl.ds(d, 1)].reshape(UP1, B)
        p1 = a + blrow
        a_sh = jnp.concatenate(
            [jnp.full((1, B), NEG, f32), a[:U, :]], axis=0)
        p2 = a_sh + yrow
        mx = jnp.maximum(p1, p2)
        mn = jnp.minimum(p1, p2)
        a_new = mx + jnp.log1p(jnp.exp(mn - mx))
        ca = jnp.where(dbrow == d, a_new, ca)
        cb = jnp.where(dbrow == d - 1, blrow, cb)
        return a_new, ca, cb

    _, cap_a, cap_b = jax.lax.fori_loop(1, T + U + 1, body,
                                        (a0, cap_a, cap_b))
    final = cap_a + cap_b
    sel = (u_iota == ul_ref[...]).astype(f32)
    out_ref[...] = (-jnp.sum(final * sel)).reshape(1, 1)


def kernel(x, x_lens, y_padded, y_lens, W_enc, b_enc, embed, W_encp, b_encp,
           W_decp, b_decp, W_out, b_out):
    sos = jnp.concatenate(
        [jnp.full((B, 1), BLANK, dtype=y_padded.dtype), y_padded],
        axis=1).astype(jnp.int32).reshape(B, 1, UP1)

    grid = (B, NT)
    bl_skew, y_skew = pl.pallas_call(
        _stage1_kernel,
        grid=grid,
        in_specs=[
            pl.BlockSpec((1, TT, F), lambda b, t: (b, t, 0)),
            pl.BlockSpec((1, 1, UP1), lambda b, t: (b, 0, 0)),
            pl.BlockSpec((F, ENC), lambda b, t: (0, 0)),
            pl.BlockSpec((1, ENC), lambda b, t: (0, 0)),
            pl.BlockSpec((V, DEC), lambda b, t: (0, 0)),
            pl.BlockSpec((ENC, JOIN), lambda b, t: (0, 0)),
            pl.BlockSpec((1, JOIN), lambda b, t: (0, 0)),
            pl.BlockSpec((DEC, JOIN), lambda b, t: (0, 0)),
            pl.BlockSpec((1, JOIN), lambda b, t: (0, 0)),
            pl.BlockSpec((JOIN, V), lambda b, t: (0, 0)),
            pl.BlockSpec((1, V), lambda b, t: (0, 0)),
        ],
        out_specs=[
            pl.BlockSpec((1, D2, UP1), lambda b, t: (b, 0, 0)),
            pl.BlockSpec((1, D2, UP1), lambda b, t: (b, 0, 0)),
        ],
        out_shape=[
            jax.ShapeDtypeStruct((B, D2, UP1), f32),
            jax.ShapeDtypeStruct((B, D2, UP1), f32),
        ],
        scratch_shapes=[
            pltpu.VMEM((UP1, JOIN), bf16),
            pltpu.VMEM((JOIN, 36), bf16),
            pltpu.VMEM((1, 36), f32),
            pltpu.VMEM((1, V), bf16),
            pltpu.VMEM((TT, UP1), f32),
            pltpu.VMEM((TT, UP1), f32),
            pltpu.VMEM((TT, UP1), f32),
            pltpu.VMEM((JOIN, V + 12), bf16),
        ],
        compiler_params=pltpu.CompilerParams(
            dimension_semantics=("arbitrary", "arbitrary")),
    )(x.astype(bf16), sos, W_enc.astype(bf16), b_enc.reshape(1, ENC),
      embed.astype(bf16), W_encp.astype(bf16), b_encp.reshape(1, JOIN),
      W_decp.astype(bf16), b_decp.reshape(1, JOIN), W_out.astype(bf16),
      b_out.reshape(1, V))

    nll = pl.pallas_call(
        _dp_kernel,
        out_shape=jax.ShapeDtypeStruct((1, 1), f32),
        scratch_shapes=[
            pltpu.VMEM((D2, UP1, B), f32),
            pltpu.VMEM((D2, UP1, B), f32),
        ],
    )(bl_skew, y_skew,
      x_lens.astype(jnp.int32).reshape(1, B),
      y_lens.astype(jnp.int32).reshape(1, B))
    return nll.reshape(())
